# baseline (speedup 1.0000x reference)
import functools

import jax
import jax.numpy as jnp
from jax import lax
from jax.experimental import pallas as pl
from jax.experimental.pallas import tpu as pltpu
from jax.experimental.pallas import tpu_sc as plsc

N = 10000
E = 160000
DN = 128
DE = 16
DL = 16
HF = 32
DH = 128
MP_STEPS = 3
POOL_STEPS = 3

NC = 2
NS = 16
NW = NC * NS
CH = 128
NCH = 40
EW = CH * NCH
EP = NW * EW
DK = DE + DL + 1
DKP = 40
EB = 4096
HCH = NCH // 2
EH = HCH * CH

_sc_mesh = plsc.VectorSubcoreMesh(core_axis_name="c", subcore_axis_name="s")


def _embed_body(a_ref, w_ref, b_ref, o_ref):
    o_ref[...] = jnp.maximum(
        jnp.dot(a_ref[...], w_ref[...], preferred_element_type=jnp.float32)
        + b_ref[...], 0.0)


def _embed(node_attr, w, b):
    return pl.pallas_call(
        _embed_body,
        out_shape=jax.ShapeDtypeStruct((N, HF), jnp.float32),
    )(node_attr, w, b)


def _ef_body(ea_ref, el_ref, m_ref, mu_ref, sd_ref, o_ref):
    el = el_ref[...]
    le = jnp.exp(-sd_ref[...] * jnp.square(el - mu_ref[...]))
    ones = jnp.ones_like(el)
    zeros = jnp.zeros((EB, DKP - DK), jnp.float32)
    ef = jnp.concatenate([ea_ref[...], le, ones, zeros], axis=1) * m_ref[...]
    o_ref[...] = lax.transpose(ef, (1, 0))


def _edge_features(eaP, elP, maskP, mu, sd):
    ne = eaP.shape[0]
    grid = ne // EB
    return pl.pallas_call(
        _ef_body,
        grid=(grid,),
        in_specs=[
            pl.BlockSpec((EB, DE), lambda i: (i, 0)),
            pl.BlockSpec((EB, 1), lambda i: (i, 0)),
            pl.BlockSpec((EB, 1), lambda i: (i, 0)),
            pl.BlockSpec((1, DL), lambda i: (0, 0)),
            pl.BlockSpec((1, DL), lambda i: (0, 0)),
        ],
        out_specs=pl.BlockSpec((DKP, EB), lambda i: (0, i)),
        out_shape=jax.ShapeDtypeStruct((DKP, ne), jnp.float32),
    )(eaP, elP, maskP, mu, sd)


def _msg_body(xg_ref, ef_ref, u_ref, o_ref, g_ref):
    xg_t = lax.transpose(xg_ref[...], (1, 0))
    ef = ef_ref[...]
    for k in range(DKP):
        g_ref[k * HF:(k + 1) * HF, :] = xg_t * ef[k:k + 1, :]
    o_ref[...] = lax.dot_general(
        g_ref[...], u_ref[...], (((0,), (0,)), ((), ())),
        preferred_element_type=jnp.float32)


def _messages(xg, ef, U):
    ne = xg.shape[0]
    grid = ne // EB
    return pl.pallas_call(
        _msg_body,
        grid=(grid,),
        in_specs=[
            pl.BlockSpec((EB, HF), lambda i: (i, 0)),
            pl.BlockSpec((DKP, EB), lambda i: (0, i)),
            pl.BlockSpec((DKP * HF, HF), lambda i: (0, 0)),
        ],
        out_specs=pl.BlockSpec((EB, HF), lambda i: (i, 0)),
        out_shape=jax.ShapeDtypeStruct((ne, HF), jnp.float32),
        scratch_shapes=[pltpu.VMEM((DKP * HF, EB), jnp.float32)],
    )(xg, ef, U)


def _gru_body(pa_ref, pb_ref, x_ref, bc_ref, wi_ref, wh_ref, bi_ref, bh_ref,
              o_ref):
    conv = pa_ref[0] + pa_ref[1] + pb_ref[0] + pb_ref[1] + bc_ref[...]
    xa = jnp.maximum(conv, 0.0)
    x = x_ref[...]
    gi = jnp.dot(xa, wi_ref[...], preferred_element_type=jnp.float32) + bi_ref[...]
    gh = jnp.dot(x, wh_ref[...], preferred_element_type=jnp.float32) + bh_ref[...]
    r = jax.nn.sigmoid(gi[:, :HF] + gh[:, :HF])
    z = jax.nn.sigmoid(gi[:, HF:2 * HF] + gh[:, HF:2 * HF])
    n = jnp.tanh(gi[:, 2 * HF:] + r * gh[:, 2 * HF:])
    o_ref[...] = (1.0 - z) * n + z * x


def _gru(pa, pb, x, bc, wiT, whT, bi, bh):
    return pl.pallas_call(
        _gru_body,
        out_shape=jax.ShapeDtypeStruct((N, HF), jnp.float32),
    )(pa, pb, x, bc, wiT, whT, bi, bh)


def _pool_body(xf_ref, x0_ref, wi_ref, wh_ref, bi_ref, bh_ref,
               wsp_ref, bsp_ref, a_ref, o_ref):
    D = 2 * HF
    agg = jnp.concatenate([xf_ref[...], x0_ref[...]], axis=1)
    q_star = jnp.zeros((1, 4 * HF), jnp.float32)
    hl = jnp.zeros((1, D), jnp.float32)
    cl = jnp.zeros((1, D), jnp.float32)
    for _ in range(POOL_STEPS):
        gates = (jnp.dot(q_star, wi_ref[...], preferred_element_type=jnp.float32)
                 + bi_ref[...]
                 + jnp.dot(hl, wh_ref[...], preferred_element_type=jnp.float32)
                 + bh_ref[...])
        ig = jax.nn.sigmoid(gates[:, :D])
        fg = jax.nn.sigmoid(gates[:, D:2 * D])
        gg = jnp.tanh(gates[:, 2 * D:3 * D])
        og = jax.nn.sigmoid(gates[:, 3 * D:])
        cl = fg * cl + ig * gg
        hl = og * jnp.tanh(cl)
        e = jnp.sum(agg * hl, axis=1, keepdims=True)
        m = jnp.max(e, axis=0, keepdims=True)
        ex = jnp.exp(e - m)
        alpha = ex / jnp.sum(ex, axis=0, keepdims=True)
        readout = jnp.sum(alpha * agg, axis=0, keepdims=True)
        q_star = jnp.concatenate([hl, readout], axis=1)
    out = jnp.dot(q_star, wsp_ref[...], preferred_element_type=jnp.float32) \
        + bsp_ref[...]
    o_ref[...] = jnp.where(out >= 0.0, out, a_ref[...] * out)


def _pool(xf, x0, wiT, whT, bi, bh, wsp, bsp, a):
    return pl.pallas_call(
        _pool_body,
        out_shape=jax.ShapeDtypeStruct((1, DH), jnp.float32),
    )(xf, x0, wiT, whT, bi, bh, wsp, bsp, a)


def _make_gather(ne):
    ew = ne // NW
    nch = ew // CH

    @functools.partial(
        pl.kernel,
        out_type=jax.ShapeDtypeStruct((ne, HF), jnp.float32),
        mesh=_sc_mesh,
        scratch_types=[
            pltpu.VMEM_SHARED((N, HF), jnp.float32),
            pltpu.VMEM((ew,), jnp.int32),
            pltpu.VMEM((ew, HF), jnp.float32),
            pltpu.SemaphoreType.DMA,
        ],
        compiler_params=pltpu.CompilerParams(use_tc_tiling_on_sc=False),
    )
    def g(x_hbm, src_hbm, out_hbm, x_sh, idx_v, rows_v, sem):
        cid = lax.axis_index("c")
        sid = lax.axis_index("s")
        wid = cid * NS + sid
        base = wid * ew
        pltpu.sync_copy(src_hbm.at[pl.ds(base, ew)], idx_v)

        @pl.when(sid == 0)
        def _():
            pltpu.sync_copy(x_hbm, x_sh)

        plsc.subcore_barrier()
        cps = [
            pltpu.async_copy(
                x_sh.at[idx_v.at[pl.ds(c * CH, CH)]],
                rows_v.at[pl.ds(c * CH, CH)], sem)
            for c in range(nch)
        ]
        for cp in cps:
            cp.wait()
        pltpu.sync_copy(rows_v, out_hbm.at[pl.ds(base, ew)])

    return g


def _make_scatter(ne):
    ew = ne // NW
    nch = ew // CH

    @functools.partial(
        pl.kernel,
        out_type=jax.ShapeDtypeStruct((NC, N, HF), jnp.float32),
        mesh=_sc_mesh,
        scratch_types=[
            pltpu.VMEM_SHARED((N, HF), jnp.float32),
            pltpu.VMEM((nch, CH), jnp.int32),
            pltpu.VMEM((ew, HF), jnp.float32),
            pltpu.SemaphoreType.DMA,
        ],
        compiler_params=pltpu.CompilerParams(use_tc_tiling_on_sc=False),
    )
    def s(msg_hbm, dst_hbm, zeros_hbm, out_hbm, acc_sh, idx_v, msg_v, sem):
        cid = lax.axis_index("c")
        sid = lax.axis_index("s")
        wid = cid * NS + sid
        pltpu.sync_copy(dst_hbm.at[pl.ds(wid * nch, nch)], idx_v)

        @pl.when(sid == 0)
        def _():
            pltpu.sync_copy(zeros_hbm, acc_sh)

        plsc.subcore_barrier()
        pltpu.sync_copy(msg_hbm.at[pl.ds(wid * ew, ew)], msg_v)
        cps = [
            pltpu.async_copy(
                msg_v.at[pl.ds(c * CH, CH)],
                acc_sh.at[idx_v.at[c]], sem, add=True)
            for c in range(nch)
        ]
        for cp in cps:
            cp.wait()
        plsc.subcore_barrier()

        @pl.when(sid == 0)
        def _():
            pltpu.sync_copy(acc_sh, out_hbm.at[cid])

    return s


EPH = EP // 2
_gather_h = _make_gather(EPH)
_scatter_h = _make_scatter(EPH)


def kernel(node_attr, edge_attr, edge_length, edge_index, W_node, b_node,
           rbf_mean, rbf_std, W_bond, b_bond, bias_conv, gru_W_ih, gru_W_hh,
           gru_b_ih, gru_b_hh, lstm_W_ih0, lstm_W_hh0, lstm_b_ih0, lstm_b_hh0,
           W_sp, b_sp, prelu_a):
    pad = EP - E
    eaP = jnp.pad(edge_attr, ((0, pad), (0, 0)))
    elP = jnp.pad(edge_length, (0, pad)).reshape(EP, 1)
    maskP = jnp.pad(jnp.ones((E, 1), jnp.float32), ((0, pad), (0, 0)))
    srcP = jnp.pad(edge_index[0], (0, pad))
    dstP = jnp.pad(edge_index[1], (0, pad)).reshape(EP // CH, CH)
    zerosN = jnp.zeros((N, HF), jnp.float32)

    U = jnp.pad(jnp.concatenate([W_bond, b_bond[None, :]], axis=0),
                ((0, DKP - DK), (0, 0))).reshape(DKP * HF, HF)

    x0 = _embed(node_attr, W_node, b_node.reshape(1, HF))
    mu = rbf_mean.reshape(1, DL)
    sd = rbf_std.reshape(1, DL)
    efA = _edge_features(eaP[:EPH], elP[:EPH], maskP[:EPH], mu, sd)
    efB = _edge_features(eaP[EPH:], elP[EPH:], maskP[EPH:], mu, sd)
    srcA, srcB = srcP[:EPH], srcP[EPH:]
    dstA, dstB = dstP[:EPH // CH], dstP[EPH // CH:]

    wiT = gru_W_ih.T
    whT = gru_W_hh.T
    bi = gru_b_ih.reshape(1, 3 * HF)
    bh = gru_b_hh.reshape(1, 3 * HF)

    x = x0
    for _ in range(MP_STEPS):
        xgA = _gather_h(x, srcA)
        msgA = _messages(xgA, efA, U)
        xgB = _gather_h(x, srcB)
        msgB = _messages(xgB, efB, U)
        pA = _scatter_h(msgA, dstA, zerosN)
        pB = _scatter_h(msgB, dstB, zerosN)
        x = _gru(pA, pB, x, bias_conv.reshape(1, HF), wiT, whT, bi, bh)

    out = _pool(x, x0, lstm_W_ih0.T, lstm_W_hh0.T,
                lstm_b_ih0.reshape(1, 4 * 2 * HF), lstm_b_hh0.reshape(1, 4 * 2 * HF),
                W_sp, b_sp.reshape(1, DH), prelu_a.reshape(1, 1))
    return out


# speedup vs baseline: 1.0670x; 1.0670x over previous
import functools

import jax
import jax.numpy as jnp
from jax import lax
from jax.experimental import pallas as pl
from jax.experimental.pallas import tpu as pltpu
from jax.experimental.pallas import tpu_sc as plsc

N = 10000
E = 160000
DN = 128
DE = 16
DL = 16
HF = 32
DH = 128
MP_STEPS = 3
POOL_STEPS = 3

NC = 2
NS = 16
NW = NC * NS
CH = 128
NCH = 40
EW = CH * NCH
EP = NW * EW
DK = DE + DL + 1
DKP = 40
EB = 4096
HCH = NCH // 2
EH = HCH * CH

_sc_mesh = plsc.VectorSubcoreMesh(core_axis_name="c", subcore_axis_name="s")


def _embed_body(a_ref, w_ref, b_ref, o_ref):
    o_ref[...] = jnp.maximum(
        jnp.dot(a_ref[...], w_ref[...], preferred_element_type=jnp.float32)
        + b_ref[...], 0.0)


def _embed(node_attr, w, b):
    return pl.pallas_call(
        _embed_body,
        out_shape=jax.ShapeDtypeStruct((N, HF), jnp.float32),
    )(node_attr, w, b)


def _ef_body(ea_ref, el_ref, m_ref, mu_ref, sd_ref, o_ref):
    el = el_ref[...]
    le = jnp.exp(-sd_ref[...] * jnp.square(el - mu_ref[...]))
    ones = jnp.ones_like(el)
    zeros = jnp.zeros((EB, DKP - DK), jnp.float32)
    ef = jnp.concatenate([ea_ref[...], le, ones, zeros], axis=1) * m_ref[...]
    o_ref[...] = lax.transpose(ef, (1, 0))


def _edge_features(eaP, elP, maskP, mu, sd):
    ne = eaP.shape[0]
    grid = ne // EB
    return pl.pallas_call(
        _ef_body,
        grid=(grid,),
        in_specs=[
            pl.BlockSpec((EB, DE), lambda i: (i, 0)),
            pl.BlockSpec((EB, 1), lambda i: (i, 0)),
            pl.BlockSpec((EB, 1), lambda i: (i, 0)),
            pl.BlockSpec((1, DL), lambda i: (0, 0)),
            pl.BlockSpec((1, DL), lambda i: (0, 0)),
        ],
        out_specs=pl.BlockSpec((DKP, EB), lambda i: (0, i)),
        out_shape=jax.ShapeDtypeStruct((DKP, ne), jnp.float32),
    )(eaP, elP, maskP, mu, sd)


def _msg_body(xg_ref, ef_ref, u_ref, o_ref, g_ref):
    xg_t = lax.transpose(xg_ref[...], (1, 0))
    ef = ef_ref[...]
    for k in range(DK):
        g_ref[k * HF:(k + 1) * HF, :] = xg_t * ef[k:k + 1, :]
    o_ref[...] = lax.dot_general(
        g_ref[...], u_ref[...], (((0,), (0,)), ((), ())),
        preferred_element_type=jnp.float32)


def _messages(xg, ef, U):
    ne = xg.shape[0]
    grid = ne // EB
    return pl.pallas_call(
        _msg_body,
        grid=(grid,),
        in_specs=[
            pl.BlockSpec((EB, HF), lambda i: (i, 0)),
            pl.BlockSpec((DKP, EB), lambda i: (0, i)),
            pl.BlockSpec((DK * HF, HF), lambda i: (0, 0)),
        ],
        out_specs=pl.BlockSpec((EB, HF), lambda i: (i, 0)),
        out_shape=jax.ShapeDtypeStruct((ne, HF), jnp.float32),
        scratch_shapes=[pltpu.VMEM((DK * HF, EB), jnp.float32)],
    )(xg, ef, U)


def _gru_body(p_ref, x_ref, bc_ref, wi_ref, wh_ref, bi_ref, bh_ref, o_ref):
    conv = p_ref[0] + p_ref[1] + bc_ref[...]
    xa = jnp.maximum(conv, 0.0)
    x = x_ref[...]
    gi = jnp.dot(xa, wi_ref[...], preferred_element_type=jnp.float32) + bi_ref[...]
    gh = jnp.dot(x, wh_ref[...], preferred_element_type=jnp.float32) + bh_ref[...]
    r = jax.nn.sigmoid(gi[:, :HF] + gh[:, :HF])
    z = jax.nn.sigmoid(gi[:, HF:2 * HF] + gh[:, HF:2 * HF])
    n = jnp.tanh(gi[:, 2 * HF:] + r * gh[:, 2 * HF:])
    o_ref[...] = (1.0 - z) * n + z * x


def _gru(parts, x, bc, wiT, whT, bi, bh):
    return pl.pallas_call(
        _gru_body,
        out_shape=jax.ShapeDtypeStruct((N, HF), jnp.float32),
    )(parts, x, bc, wiT, whT, bi, bh)


def _pool_body(xf_ref, x0_ref, wi_ref, wh_ref, bi_ref, bh_ref,
               wsp_ref, bsp_ref, a_ref, o_ref):
    D = 2 * HF
    agg = jnp.concatenate([xf_ref[...], x0_ref[...]], axis=1)
    q_star = jnp.zeros((1, 4 * HF), jnp.float32)
    hl = jnp.zeros((1, D), jnp.float32)
    cl = jnp.zeros((1, D), jnp.float32)
    for _ in range(POOL_STEPS):
        gates = (jnp.dot(q_star, wi_ref[...], preferred_element_type=jnp.float32)
                 + bi_ref[...]
                 + jnp.dot(hl, wh_ref[...], preferred_element_type=jnp.float32)
                 + bh_ref[...])
        ig = jax.nn.sigmoid(gates[:, :D])
        fg = jax.nn.sigmoid(gates[:, D:2 * D])
        gg = jnp.tanh(gates[:, 2 * D:3 * D])
        og = jax.nn.sigmoid(gates[:, 3 * D:])
        cl = fg * cl + ig * gg
        hl = og * jnp.tanh(cl)
        e = jnp.sum(agg * hl, axis=1, keepdims=True)
        m = jnp.max(e, axis=0, keepdims=True)
        ex = jnp.exp(e - m)
        alpha = ex / jnp.sum(ex, axis=0, keepdims=True)
        readout = jnp.sum(alpha * agg, axis=0, keepdims=True)
        q_star = jnp.concatenate([hl, readout], axis=1)
    out = jnp.dot(q_star, wsp_ref[...], preferred_element_type=jnp.float32) \
        + bsp_ref[...]
    o_ref[...] = jnp.where(out >= 0.0, out, a_ref[...] * out)


def _pool(xf, x0, wiT, whT, bi, bh, wsp, bsp, a):
    return pl.pallas_call(
        _pool_body,
        out_shape=jax.ShapeDtypeStruct((1, DH), jnp.float32),
    )(xf, x0, wiT, whT, bi, bh, wsp, bsp, a)


def _make_gather(ne):
    ew = ne // NW
    nch = ew // CH

    @functools.partial(
        pl.kernel,
        out_type=jax.ShapeDtypeStruct((ne, HF), jnp.float32),
        mesh=_sc_mesh,
        scratch_types=[
            pltpu.VMEM_SHARED((N, HF), jnp.float32),
            pltpu.VMEM((ew,), jnp.int32),
            pltpu.VMEM((min(ew, EH), HF), jnp.float32),
            pltpu.SemaphoreType.DMA,
        ],
        compiler_params=pltpu.CompilerParams(use_tc_tiling_on_sc=False),
    )
    def g(x_hbm, src_hbm, out_hbm, x_sh, idx_v, rows_v, sem):
        cid = lax.axis_index("c")
        sid = lax.axis_index("s")
        wid = cid * NS + sid
        base = wid * ew
        pltpu.sync_copy(src_hbm.at[pl.ds(base, ew)], idx_v)

        @pl.when(sid == 0)
        def _():
            pltpu.sync_copy(x_hbm, x_sh)

        plsc.subcore_barrier()
        bs = min(nch, HCH)
        for b in range(nch // bs):
            hb = b * bs * CH
            cps = [
                pltpu.async_copy(
                    x_sh.at[idx_v.at[pl.ds(hb + c * CH, CH)]],
                    rows_v.at[pl.ds(c * CH, CH)], sem)
                for c in range(bs)
            ]
            for cp in cps:
                cp.wait()
            pltpu.sync_copy(rows_v, out_hbm.at[pl.ds(base + hb, bs * CH)])

    return g


def _make_scatter(ne):
    ew = ne // NW
    nch = ew // CH

    @functools.partial(
        pl.kernel,
        out_type=jax.ShapeDtypeStruct((NC, N, HF), jnp.float32),
        mesh=_sc_mesh,
        scratch_types=[
            pltpu.VMEM_SHARED((N, HF), jnp.float32),
            pltpu.VMEM((nch, CH), jnp.int32),
            pltpu.VMEM((min(ew, EH), HF), jnp.float32),
            pltpu.SemaphoreType.DMA,
        ],
        compiler_params=pltpu.CompilerParams(use_tc_tiling_on_sc=False),
    )
    def s(msg_hbm, dst_hbm, zeros_hbm, out_hbm, acc_sh, idx_v, msg_v, sem):
        cid = lax.axis_index("c")
        sid = lax.axis_index("s")
        wid = cid * NS + sid
        pltpu.sync_copy(dst_hbm.at[pl.ds(wid * nch, nch)], idx_v)

        @pl.when(sid == 0)
        def _():
            pltpu.sync_copy(zeros_hbm, acc_sh)

        plsc.subcore_barrier()
        bs = min(nch, HCH)
        for b in range(nch // bs):
            pltpu.sync_copy(msg_hbm.at[pl.ds(wid * ew + b * bs * CH, bs * CH)],
                            msg_v)
            cps = [
                pltpu.async_copy(
                    msg_v.at[pl.ds(c * CH, CH)],
                    acc_sh.at[idx_v.at[b * bs + c]], sem, add=True)
                for c in range(bs)
            ]
            for cp in cps:
                cp.wait()
        plsc.subcore_barrier()

        @pl.when(sid == 0)
        def _():
            pltpu.sync_copy(acc_sh, out_hbm.at[cid])

    return s


_gather_f = _make_gather(EP)
_scatter_f = _make_scatter(EP)


def kernel(node_attr, edge_attr, edge_length, edge_index, W_node, b_node,
           rbf_mean, rbf_std, W_bond, b_bond, bias_conv, gru_W_ih, gru_W_hh,
           gru_b_ih, gru_b_hh, lstm_W_ih0, lstm_W_hh0, lstm_b_ih0, lstm_b_hh0,
           W_sp, b_sp, prelu_a):
    pad = EP - E
    eaP = jnp.pad(edge_attr, ((0, pad), (0, 0)))
    elP = jnp.pad(edge_length, (0, pad)).reshape(EP, 1)
    maskP = jnp.pad(jnp.ones((E, 1), jnp.float32), ((0, pad), (0, 0)))
    srcP = jnp.pad(edge_index[0], (0, pad))
    dstP = jnp.pad(edge_index[1], (0, pad)).reshape(EP // CH, CH)
    zerosN = jnp.zeros((N, HF), jnp.float32)

    U = jnp.concatenate([W_bond, b_bond[None, :]],
                        axis=0).reshape(DK * HF, HF)

    x0 = _embed(node_attr, W_node, b_node.reshape(1, HF))
    ef = _edge_features(eaP, elP, maskP, rbf_mean.reshape(1, DL),
                        rbf_std.reshape(1, DL))

    wiT = gru_W_ih.T
    whT = gru_W_hh.T
    bi = gru_b_ih.reshape(1, 3 * HF)
    bh = gru_b_hh.reshape(1, 3 * HF)

    x = x0
    for _ in range(MP_STEPS):
        xg = _gather_f(x, srcP)
        msg = _messages(xg, ef, U)
        parts = _scatter_f(msg, dstP, zerosN)
        x = _gru(parts, x, bias_conv.reshape(1, HF), wiT, whT, bi, bh)

    out = _pool(x, x0, lstm_W_ih0.T, lstm_W_hh0.T,
                lstm_b_ih0.reshape(1, 4 * 2 * HF), lstm_b_hh0.reshape(1, 4 * 2 * HF),
                W_sp, b_sp.reshape(1, DH), prelu_a.reshape(1, 1))
    return out


# speedup vs baseline: 1.0793x; 1.0116x over previous
import functools

import jax
import jax.numpy as jnp
from jax import lax
from jax.experimental import pallas as pl
from jax.experimental.pallas import tpu as pltpu
from jax.experimental.pallas import tpu_sc as plsc

N = 10000
E = 160000
DN = 128
DE = 16
DL = 16
HF = 32
DH = 128
MP_STEPS = 3
POOL_STEPS = 3

NC = 2
NS = 16
NW = NC * NS
CH = 128
NCH = 40
EW = CH * NCH
EP = NW * EW
DK = DE + DL + 1
DKP = 40
EB = 8192
HCH = NCH // 2
EH = HCH * CH

_sc_mesh = plsc.VectorSubcoreMesh(core_axis_name="c", subcore_axis_name="s")


def _embed_body(a_ref, w_ref, b_ref, o_ref):
    o_ref[...] = jnp.maximum(
        jnp.dot(a_ref[...], w_ref[...], preferred_element_type=jnp.float32)
        + b_ref[...], 0.0)


def _embed(node_attr, w, b):
    return pl.pallas_call(
        _embed_body,
        out_shape=jax.ShapeDtypeStruct((N, HF), jnp.float32),
    )(node_attr, w, b)


def _ef_body(ea_ref, el_ref, m_ref, mu_ref, sd_ref, o_ref):
    el = el_ref[...]
    le = jnp.exp(-sd_ref[...] * jnp.square(el - mu_ref[...]))
    ones = jnp.ones_like(el)
    zeros = jnp.zeros((EB, DKP - DK), jnp.float32)
    ef = jnp.concatenate([ea_ref[...], le, ones, zeros], axis=1) * m_ref[...]
    o_ref[...] = lax.transpose(ef, (1, 0))


def _edge_features(eaP, elP, maskP, mu, sd):
    ne = eaP.shape[0]
    grid = ne // EB
    return pl.pallas_call(
        _ef_body,
        grid=(grid,),
        in_specs=[
            pl.BlockSpec((EB, DE), lambda i: (i, 0)),
            pl.BlockSpec((EB, 1), lambda i: (i, 0)),
            pl.BlockSpec((EB, 1), lambda i: (i, 0)),
            pl.BlockSpec((1, DL), lambda i: (0, 0)),
            pl.BlockSpec((1, DL), lambda i: (0, 0)),
        ],
        out_specs=pl.BlockSpec((DKP, EB), lambda i: (0, i)),
        out_shape=jax.ShapeDtypeStruct((DKP, ne), jnp.float32),
    )(eaP, elP, maskP, mu, sd)


def _msg_body(xg_ref, ef_ref, u_ref, o_ref, g_ref):
    xg_t = lax.transpose(xg_ref[...], (1, 0))
    ef = ef_ref[...]
    for k in range(DK):
        g_ref[k * HF:(k + 1) * HF, :] = xg_t * ef[k:k + 1, :]
    o_ref[...] = lax.dot_general(
        g_ref[...], u_ref[...], (((0,), (0,)), ((), ())),
        preferred_element_type=jnp.float32)


def _messages(xg, ef, U):
    ne = xg.shape[0]
    grid = ne // EB
    return pl.pallas_call(
        _msg_body,
        grid=(grid,),
        in_specs=[
            pl.BlockSpec((EB, HF), lambda i: (i, 0)),
            pl.BlockSpec((DKP, EB), lambda i: (0, i)),
            pl.BlockSpec((DK * HF, HF), lambda i: (0, 0)),
        ],
        out_specs=pl.BlockSpec((EB, HF), lambda i: (i, 0)),
        out_shape=jax.ShapeDtypeStruct((ne, HF), jnp.float32),
        scratch_shapes=[pltpu.VMEM((DK * HF, EB), jnp.float32)],
    )(xg, ef, U)


def _gru_body(p_ref, x_ref, bc_ref, wi_ref, wh_ref, bi_ref, bh_ref, o_ref):
    conv = p_ref[0] + p_ref[1] + bc_ref[...]
    xa = jnp.maximum(conv, 0.0)
    x = x_ref[...]
    gi = jnp.dot(xa, wi_ref[...], preferred_element_type=jnp.float32) + bi_ref[...]
    gh = jnp.dot(x, wh_ref[...], preferred_element_type=jnp.float32) + bh_ref[...]
    r = jax.nn.sigmoid(gi[:, :HF] + gh[:, :HF])
    z = jax.nn.sigmoid(gi[:, HF:2 * HF] + gh[:, HF:2 * HF])
    n = jnp.tanh(gi[:, 2 * HF:] + r * gh[:, 2 * HF:])
    o_ref[...] = (1.0 - z) * n + z * x


def _gru(parts, x, bc, wiT, whT, bi, bh):
    return pl.pallas_call(
        _gru_body,
        out_shape=jax.ShapeDtypeStruct((N, HF), jnp.float32),
    )(parts, x, bc, wiT, whT, bi, bh)


def _pool_body(xf_ref, x0_ref, wi_ref, wh_ref, bi_ref, bh_ref,
               wsp_ref, bsp_ref, a_ref, o_ref):
    D = 2 * HF
    agg = jnp.concatenate([xf_ref[...], x0_ref[...]], axis=1)
    q_star = jnp.zeros((1, 4 * HF), jnp.float32)
    hl = jnp.zeros((1, D), jnp.float32)
    cl = jnp.zeros((1, D), jnp.float32)
    for _ in range(POOL_STEPS):
        gates = (jnp.dot(q_star, wi_ref[...], preferred_element_type=jnp.float32)
                 + bi_ref[...]
                 + jnp.dot(hl, wh_ref[...], preferred_element_type=jnp.float32)
                 + bh_ref[...])
        ig = jax.nn.sigmoid(gates[:, :D])
        fg = jax.nn.sigmoid(gates[:, D:2 * D])
        gg = jnp.tanh(gates[:, 2 * D:3 * D])
        og = jax.nn.sigmoid(gates[:, 3 * D:])
        cl = fg * cl + ig * gg
        hl = og * jnp.tanh(cl)
        e = jnp.sum(agg * hl, axis=1, keepdims=True)
        m = jnp.max(e, axis=0, keepdims=True)
        ex = jnp.exp(e - m)
        alpha = ex / jnp.sum(ex, axis=0, keepdims=True)
        readout = jnp.sum(alpha * agg, axis=0, keepdims=True)
        q_star = jnp.concatenate([hl, readout], axis=1)
    out = jnp.dot(q_star, wsp_ref[...], preferred_element_type=jnp.float32) \
        + bsp_ref[...]
    o_ref[...] = jnp.where(out >= 0.0, out, a_ref[...] * out)


def _pool(xf, x0, wiT, whT, bi, bh, wsp, bsp, a):
    return pl.pallas_call(
        _pool_body,
        out_shape=jax.ShapeDtypeStruct((1, DH), jnp.float32),
    )(xf, x0, wiT, whT, bi, bh, wsp, bsp, a)


def _make_gather(ne):
    ew = ne // NW
    nch = ew // CH

    @functools.partial(
        pl.kernel,
        out_type=jax.ShapeDtypeStruct((ne, HF), jnp.float32),
        mesh=_sc_mesh,
        scratch_types=[
            pltpu.VMEM_SHARED((N, HF), jnp.float32),
            pltpu.VMEM((ew,), jnp.int32),
            pltpu.VMEM((min(ew, EH), HF), jnp.float32),
            pltpu.SemaphoreType.DMA,
        ],
        compiler_params=pltpu.CompilerParams(use_tc_tiling_on_sc=False),
    )
    def g(x_hbm, src_hbm, out_hbm, x_sh, idx_v, rows_v, sem):
        cid = lax.axis_index("c")
        sid = lax.axis_index("s")
        wid = cid * NS + sid
        base = wid * ew
        pltpu.sync_copy(src_hbm.at[pl.ds(base, ew)], idx_v)

        @pl.when(sid == 0)
        def _():
            pltpu.sync_copy(x_hbm, x_sh)

        plsc.subcore_barrier()
        bs = min(nch, HCH)
        for b in range(nch // bs):
            hb = b * bs * CH
            cps = [
                pltpu.async_copy(
                    x_sh.at[idx_v.at[pl.ds(hb + c * CH, CH)]],
                    rows_v.at[pl.ds(c * CH, CH)], sem)
                for c in range(bs)
            ]
            for cp in cps:
                cp.wait()
            pltpu.sync_copy(rows_v, out_hbm.at[pl.ds(base + hb, bs * CH)])

    return g


def _make_scatter(ne):
    ew = ne // NW
    nch = ew // CH

    @functools.partial(
        pl.kernel,
        out_type=jax.ShapeDtypeStruct((NC, N, HF), jnp.float32),
        mesh=_sc_mesh,
        scratch_types=[
            pltpu.VMEM_SHARED((N, HF), jnp.float32),
            pltpu.VMEM((nch, CH), jnp.int32),
            pltpu.VMEM((min(ew, EH), HF), jnp.float32),
            pltpu.SemaphoreType.DMA,
        ],
        compiler_params=pltpu.CompilerParams(use_tc_tiling_on_sc=False),
    )
    def s(msg_hbm, dst_hbm, zeros_hbm, out_hbm, acc_sh, idx_v, msg_v, sem):
        cid = lax.axis_index("c")
        sid = lax.axis_index("s")
        wid = cid * NS + sid
        pltpu.sync_copy(dst_hbm.at[pl.ds(wid * nch, nch)], idx_v)

        @pl.when(sid == 0)
        def _():
            pltpu.sync_copy(zeros_hbm, acc_sh)

        plsc.subcore_barrier()
        bs = min(nch, HCH)
        for b in range(nch // bs):
            pltpu.sync_copy(msg_hbm.at[pl.ds(wid * ew + b * bs * CH, bs * CH)],
                            msg_v)
            cps = [
                pltpu.async_copy(
                    msg_v.at[pl.ds(c * CH, CH)],
                    acc_sh.at[idx_v.at[b * bs + c]], sem, add=True)
                for c in range(bs)
            ]
            for cp in cps:
                cp.wait()
        plsc.subcore_barrier()

        @pl.when(sid == 0)
        def _():
            pltpu.sync_copy(acc_sh, out_hbm.at[cid])

    return s


_gather_f = _make_gather(EP)
_scatter_f = _make_scatter(EP)


def kernel(node_attr, edge_attr, edge_length, edge_index, W_node, b_node,
           rbf_mean, rbf_std, W_bond, b_bond, bias_conv, gru_W_ih, gru_W_hh,
           gru_b_ih, gru_b_hh, lstm_W_ih0, lstm_W_hh0, lstm_b_ih0, lstm_b_hh0,
           W_sp, b_sp, prelu_a):
    pad = EP - E
    eaP = jnp.pad(edge_attr, ((0, pad), (0, 0)))
    elP = jnp.pad(edge_length, (0, pad)).reshape(EP, 1)
    maskP = jnp.pad(jnp.ones((E, 1), jnp.float32), ((0, pad), (0, 0)))
    srcP = jnp.pad(edge_index[0], (0, pad))
    dstP = jnp.pad(edge_index[1], (0, pad)).reshape(EP // CH, CH)
    zerosN = jnp.zeros((N, HF), jnp.float32)

    U = jnp.concatenate([W_bond, b_bond[None, :]],
                        axis=0).reshape(DK * HF, HF)

    x0 = _embed(node_attr, W_node, b_node.reshape(1, HF))
    ef = _edge_features(eaP, elP, maskP, rbf_mean.reshape(1, DL),
                        rbf_std.reshape(1, DL))

    wiT = gru_W_ih.T
    whT = gru_W_hh.T
    bi = gru_b_ih.reshape(1, 3 * HF)
    bh = gru_b_hh.reshape(1, 3 * HF)

    x = x0
    for _ in range(MP_STEPS):
        xg = _gather_f(x, srcP)
        msg = _messages(xg, ef, U)
        parts = _scatter_f(msg, dstP, zerosN)
        x = _gru(parts, x, bias_conv.reshape(1, HF), wiT, whT, bi, bh)

    out = _pool(x, x0, lstm_W_ih0.T, lstm_W_hh0.T,
                lstm_b_ih0.reshape(1, 4 * 2 * HF), lstm_b_hh0.reshape(1, 4 * 2 * HF),
                W_sp, b_sp.reshape(1, DH), prelu_a.reshape(1, 1))
    return out
